# Initial kernel scaffold; baseline (speedup 1.0000x reference)
import jax
import jax.numpy as jnp
from jax.experimental import pallas as pl


def kernel(inputs, edge_index, W_self1, W_neigh1, b1, W_self2, W_neigh2, b2):
    raise NotImplementedError("write your pallas kernel here")


# speedup vs baseline: 4.8422x; 4.8422x over previous
import functools

import jax
import jax.numpy as jnp
from jax import lax
from jax.experimental import pallas as pl
from jax.experimental.pallas import tpu as pltpu
from jax.experimental.pallas import tpu_sc as plsc

N = 10000
E = 320000
D = 128

NC = 2
NS = 16
NP = 10240
E_PER_CORE = E // NC
E_PER_WORKER = E_PER_CORE // NS
B = 80
ITERS = E_PER_WORKER // B
WROWS = NP // NS
NZ = WROWS // B


def _fill_iota(idx_v, base):
    lanes = lax.iota(jnp.int32, 16)
    for g in range(B // 16):
        idx_v[pl.ds(g * 16, 16)] = base + g * 16 + lanes


def _fill_const(buf_v, val):
    def init_row(i, _):
        for j in range(D // 16):
            buf_v[i, pl.ds(j * 16, 16)] = jnp.full((16,), val, jnp.float32)
        return 0
    lax.fori_loop(0, B, init_row, 0)


def _sc_aggregate(with_deg: bool):
    mesh = plsc.VectorSubcoreMesh(core_axis_name="c", subcore_axis_name="s")
    out_type = [jax.ShapeDtypeStruct((NC * NP, D), jnp.float32)]
    if with_deg:
        out_type.append(jax.ShapeDtypeStruct((NC * NP, D), jnp.float32))
    scratch = [
        pltpu.VMEM((B,), jnp.int32),
        pltpu.VMEM((B,), jnp.int32),
        pltpu.VMEM((B, D), jnp.float32),
        pltpu.VMEM_SHARED((NP, D), jnp.float32),
        pltpu.SemaphoreType.DMA,
    ]

    def body(h_hbm, src_hbm, dst_hbm, *refs):
        if with_deg:
            agg_hbm, deg_hbm, sidx_v, didx_v, msg_v, agg_sh, sem = refs
        else:
            agg_hbm, sidx_v, didx_v, msg_v, agg_sh, sem = refs
        c = lax.axis_index("c")
        s = lax.axis_index("s")
        row0 = s * WROWS
        out0 = c * NP + row0
        edge0 = c * E_PER_CORE + s * E_PER_WORKER

        def zero_accumulator():
            for z in range(NZ):
                _fill_iota(didx_v, row0 + z * B)
                pltpu.sync_copy(msg_v, agg_sh.at[didx_v])

        def write_out(dst_hbm_ref):
            for z in range(NZ):
                _fill_iota(didx_v, row0 + z * B)
                pltpu.async_copy(agg_sh.at[didx_v], msg_v, sem).wait()
                pltpu.sync_copy(msg_v, dst_hbm_ref.at[pl.ds(out0 + z * B, B)])

        _fill_const(msg_v, 0.0)
        zero_accumulator()
        plsc.subcore_barrier()

        def step(g, _):
            base = edge0 + g * B
            pltpu.sync_copy(src_hbm.at[pl.ds(base, B)], sidx_v)
            pltpu.sync_copy(dst_hbm.at[pl.ds(base, B)], didx_v)
            pltpu.async_copy(h_hbm.at[sidx_v], msg_v, sem).wait()
            pltpu.sync_copy(msg_v, agg_sh.at[didx_v], add=True)
            return 0
        lax.fori_loop(0, ITERS, step, 0)
        plsc.subcore_barrier()
        write_out(agg_hbm)

        if not with_deg:
            return

        plsc.subcore_barrier()
        _fill_const(msg_v, 0.0)
        zero_accumulator()
        plsc.subcore_barrier()
        _fill_const(msg_v, 1.0)

        def step_deg(g, _):
            base = edge0 + g * B
            pltpu.sync_copy(dst_hbm.at[pl.ds(base, B)], didx_v)
            pltpu.sync_copy(msg_v, agg_sh.at[didx_v], add=True)
            return 0
        lax.fori_loop(0, ITERS, step_deg, 0)
        plsc.subcore_barrier()
        write_out(deg_hbm)

    return pl.kernel(body, out_type=tuple(out_type), mesh=mesh,
                     scratch_types=scratch)


_sc_agg_deg = _sc_aggregate(with_deg=True)
_sc_agg = _sc_aggregate(with_deg=False)

R = 1000


def _tc_layer_body(relu, x_ref, agg0_ref, agg1_ref, deg0_ref, deg1_ref,
                   ws_ref, wn_ref, b_ref, out_ref):
    deg = deg0_ref[0, :, 0:1] + deg1_ref[0, :, 0:1]
    rdeg = 1.0 / jnp.maximum(deg, 1.0)
    hn = (agg0_ref[0] + agg1_ref[0]) * rdeg
    acc = jnp.dot(x_ref[...], ws_ref[...], preferred_element_type=jnp.float32)
    acc += jnp.dot(hn, wn_ref[...], preferred_element_type=jnp.float32)
    acc += b_ref[...]
    if relu:
        acc = jnp.maximum(acc, 0.0)
    out_ref[...] = acc


def _tc_layer(x, agg, deg, w_self, w_neigh, b, relu):
    grid = (N // R,)
    return pl.pallas_call(
        functools.partial(_tc_layer_body, relu),
        grid=grid,
        in_specs=[
            pl.BlockSpec((R, D), lambda i: (i, 0)),
            pl.BlockSpec((1, R, D), lambda i: (0, i, 0)),
            pl.BlockSpec((1, R, D), lambda i: (1, i, 0)),
            pl.BlockSpec((1, R, D), lambda i: (0, i, 0)),
            pl.BlockSpec((1, R, D), lambda i: (1, i, 0)),
            pl.BlockSpec((D, D), lambda i: (0, 0)),
            pl.BlockSpec((D, D), lambda i: (0, 0)),
            pl.BlockSpec((1, D), lambda i: (0, 0)),
        ],
        out_specs=pl.BlockSpec((R, D), lambda i: (i, 0)),
        out_shape=jax.ShapeDtypeStruct((N, D), jnp.float32),
    )(x, agg, agg, deg, deg, w_self, w_neigh, b)


def kernel(inputs, edge_index, W_self1, W_neigh1, b1, W_self2, W_neigh2, b2):
    src = edge_index[0]
    dst = edge_index[1]
    b1r = b1.reshape(1, D)
    b2r = b2.reshape(1, D)

    agg1, deg = _sc_agg_deg(inputs, src, dst)
    agg1 = agg1.reshape(NC, NP, D)
    deg = deg.reshape(NC, NP, D)
    h1 = _tc_layer(inputs, agg1, deg, W_self1, W_neigh1, b1r, relu=True)
    agg2, = _sc_agg(h1, src, dst)
    agg2 = agg2.reshape(NC, NP, D)
    out = _tc_layer(h1, agg2, deg, W_self2, W_neigh2, b2r, relu=False)
    return out


# speedup vs baseline: 7.8538x; 1.6219x over previous
import functools

import jax
import jax.numpy as jnp
from jax import lax
from jax.experimental import pallas as pl
from jax.experimental.pallas import tpu as pltpu
from jax.experimental.pallas import tpu_sc as plsc

N = 10000
E = 320000
D = 128

NC = 2
NS = 16
NP = 10240
E_PER_CORE = E // NC
E_PER_WORKER = E_PER_CORE // NS
B = 80
ITERS = E_PER_WORKER // B
PAIRS = (ITERS - 1) // 2
WROWS = NP // NS
NZ = WROWS // B


def _fill_iota(idx_v, base):
    lanes = lax.iota(jnp.int32, 16)
    for g in range(B // 16):
        idx_v[pl.ds(g * 16, 16)] = base + g * 16 + lanes


def _fill_const(buf_v, val):
    def init_row(i, _):
        for j in range(D // 16):
            buf_v[i, pl.ds(j * 16, 16)] = jnp.full((16,), val, jnp.float32)
        return 0
    lax.fori_loop(0, B, init_row, 0)


def _sc_aggregate(with_deg: bool):
    mesh = plsc.VectorSubcoreMesh(core_axis_name="c", subcore_axis_name="s")
    out_type = [jax.ShapeDtypeStruct((NC * NP, D), jnp.float32)]
    if with_deg:
        out_type.append(jax.ShapeDtypeStruct((NC * NP, D), jnp.float32))
    scratch = [
        pltpu.VMEM((2, B), jnp.int32),
        pltpu.VMEM((2, B), jnp.int32),
        pltpu.VMEM((2, B, D), jnp.float32),
        pltpu.VMEM_SHARED((NP, D), jnp.float32),
        pltpu.SemaphoreType.DMA,
        pltpu.SemaphoreType.DMA,
        pltpu.SemaphoreType.DMA,
        pltpu.SemaphoreType.DMA,
    ]

    def body(h_hbm, src_hbm, dst_hbm, *refs):
        if with_deg:
            agg_hbm, deg_hbm, sidx2, didx2, msg2, agg_sh, *sems = refs
        else:
            agg_hbm, sidx2, didx2, msg2, agg_sh, *sems = refs
        semg = sems[0:2]
        sems_ = sems[2:4]
        c = lax.axis_index("c")
        s = lax.axis_index("s")
        row0 = s * WROWS
        out0 = c * NP + row0
        edge0 = c * E_PER_CORE + s * E_PER_WORKER

        def zero_accumulator():
            for z in range(NZ):
                _fill_iota(didx2.at[0], row0 + z * B)
                pltpu.sync_copy(msg2.at[0], agg_sh.at[didx2.at[0]])

        def write_out(dst_hbm_ref):
            for z in range(NZ):
                _fill_iota(didx2.at[0], row0 + z * B)
                pltpu.async_copy(agg_sh.at[didx2.at[0]], msg2.at[0],
                                 semg[0]).wait()
                pltpu.sync_copy(msg2.at[0],
                                dst_hbm_ref.at[pl.ds(out0 + z * B, B)])

        _fill_const(msg2.at[0], 0.0)
        zero_accumulator()
        plsc.subcore_barrier()

        def fire(chunk, b):
            base = edge0 + chunk * B
            pltpu.sync_copy(src_hbm.at[pl.ds(base, B)], sidx2.at[b])
            pltpu.sync_copy(dst_hbm.at[pl.ds(base, B)], didx2.at[b])
            pltpu.async_copy(h_hbm.at[sidx2.at[b]], msg2.at[b], semg[b])

        def gwait(b):
            pltpu.make_async_copy(h_hbm.at[sidx2.at[b]], msg2.at[b],
                                  semg[b]).wait()

        def sstart(b):
            pltpu.async_copy(msg2.at[b], agg_sh.at[didx2.at[b]], sems_[b],
                             add=True)

        def swait(b):
            pltpu.make_async_copy(msg2.at[b], agg_sh.at[didx2.at[b]],
                                  sems_[b]).wait()

        fire(0, 0)
        fire(1, 1)

        def pair(p, _):
            for b in range(2):
                chunk = 2 * p + b
                gwait(b)
                sstart(b)
                swait(b)
                if b == 0:
                    fire(chunk + 2, 0)
                else:
                    @pl.when(chunk + 2 < ITERS)
                    def _():
                        fire(chunk + 2, 1)
            return 0
        lax.fori_loop(0, PAIRS, pair, 0)
        gwait(0)
        sstart(0)
        swait(0)
        plsc.subcore_barrier()
        write_out(agg_hbm)

        if not with_deg:
            return

        plsc.subcore_barrier()
        _fill_const(msg2.at[0], 0.0)
        zero_accumulator()
        plsc.subcore_barrier()
        _fill_const(msg2.at[0], 1.0)

        def dfire(chunk, b):
            base = edge0 + chunk * B
            pltpu.sync_copy(dst_hbm.at[pl.ds(base, B)], didx2.at[b])
            pltpu.async_copy(msg2.at[0], agg_sh.at[didx2.at[b]], sems_[b],
                             add=True)

        def dwait(b):
            pltpu.make_async_copy(msg2.at[0], agg_sh.at[didx2.at[b]],
                                  sems_[b]).wait()

        dfire(0, 0)
        dfire(1, 1)

        def dpair(p, _):
            for b in range(2):
                chunk = 2 * p + b
                dwait(b)
                dfire(chunk, b)
            return 0
        lax.fori_loop(1, PAIRS, dpair, 0)
        dwait(0)
        dfire(ITERS - 1, 0)
        dwait(0)
        dwait(1)
        plsc.subcore_barrier()
        write_out(deg_hbm)

    return pl.kernel(body, out_type=tuple(out_type), mesh=mesh,
                     scratch_types=scratch)


_sc_agg_deg = _sc_aggregate(with_deg=True)
_sc_agg = _sc_aggregate(with_deg=False)

R = 1000


def _tc_layer_body(relu, x_ref, agg0_ref, agg1_ref, deg0_ref, deg1_ref,
                   ws_ref, wn_ref, b_ref, out_ref):
    deg = deg0_ref[0, :, 0:1] + deg1_ref[0, :, 0:1]
    rdeg = 1.0 / jnp.maximum(deg, 1.0)
    hn = (agg0_ref[0] + agg1_ref[0]) * rdeg
    acc = jnp.dot(x_ref[...], ws_ref[...], preferred_element_type=jnp.float32)
    acc += jnp.dot(hn, wn_ref[...], preferred_element_type=jnp.float32)
    acc += b_ref[...]
    if relu:
        acc = jnp.maximum(acc, 0.0)
    out_ref[...] = acc


def _tc_layer(x, agg, deg, w_self, w_neigh, b, relu):
    grid = (N // R,)
    return pl.pallas_call(
        functools.partial(_tc_layer_body, relu),
        grid=grid,
        in_specs=[
            pl.BlockSpec((R, D), lambda i: (i, 0)),
            pl.BlockSpec((1, R, D), lambda i: (0, i, 0)),
            pl.BlockSpec((1, R, D), lambda i: (1, i, 0)),
            pl.BlockSpec((1, R, D), lambda i: (0, i, 0)),
            pl.BlockSpec((1, R, D), lambda i: (1, i, 0)),
            pl.BlockSpec((D, D), lambda i: (0, 0)),
            pl.BlockSpec((D, D), lambda i: (0, 0)),
            pl.BlockSpec((1, D), lambda i: (0, 0)),
        ],
        out_specs=pl.BlockSpec((R, D), lambda i: (i, 0)),
        out_shape=jax.ShapeDtypeStruct((N, D), jnp.float32),
    )(x, agg, agg, deg, deg, w_self, w_neigh, b)


def kernel(inputs, edge_index, W_self1, W_neigh1, b1, W_self2, W_neigh2, b2):
    src = edge_index[0]
    dst = edge_index[1]
    b1r = b1.reshape(1, D)
    b2r = b2.reshape(1, D)

    agg1, deg = _sc_agg_deg(inputs, src, dst)
    agg1 = agg1.reshape(NC, NP, D)
    deg = deg.reshape(NC, NP, D)
    h1 = _tc_layer(inputs, agg1, deg, W_self1, W_neigh1, b1r, relu=True)
    agg2, = _sc_agg(h1, src, dst)
    agg2 = agg2.reshape(NC, NP, D)
    out = _tc_layer(h1, agg2, deg, W_self2, W_neigh2, b2r, relu=False)
    return out


# speedup vs baseline: 8.8189x; 1.1229x over previous
import functools

import jax
import jax.numpy as jnp
from jax import lax
from jax.experimental import pallas as pl
from jax.experimental.pallas import tpu as pltpu
from jax.experimental.pallas import tpu_sc as plsc

N = 10000
E = 320000
D = 128

NC = 2
NS = 16
NP = 10240
E_PER_CORE = E // NC
E_PER_WORKER = E_PER_CORE // NS
B = 80
ITERS = E_PER_WORKER // B
PAIRS = (ITERS - 1) // 2
WROWS = NP // NS
NZ = WROWS // B


def _fill_iota(idx_v, base):
    lanes = lax.iota(jnp.int32, 16)
    for g in range(B // 16):
        idx_v[pl.ds(g * 16, 16)] = base + g * 16 + lanes


def _fill_const(buf_v, val):
    def init_row(i, _):
        for j in range(D // 16):
            buf_v[i, pl.ds(j * 16, 16)] = jnp.full((16,), val, jnp.float32)
        return 0
    lax.fori_loop(0, B, init_row, 0)


def _sc_aggregate(with_deg: bool):
    mesh = plsc.VectorSubcoreMesh(core_axis_name="c", subcore_axis_name="s")
    out_type = [jax.ShapeDtypeStruct((NC * NP, D), jnp.float32)]
    if with_deg:
        out_type.append(jax.ShapeDtypeStruct((NC * NP, D), jnp.float32))
    scratch = [
        pltpu.VMEM((3, B), jnp.int32),
        pltpu.VMEM((3, B), jnp.int32),
        pltpu.VMEM((3, B, D), jnp.float32),
        pltpu.VMEM_SHARED((NP, D), jnp.float32),
    ] + [pltpu.SemaphoreType.DMA] * 12

    def body(h_hbm, src_hbm, dst_hbm, *refs):
        if with_deg:
            agg_hbm, deg_hbm, sidx2, didx2, msg2, agg_sh, *sems = refs
        else:
            agg_hbm, sidx2, didx2, msg2, agg_sh, *sems = refs
        semg = sems[0:3]
        sems_ = sems[3:6]
        semis = sems[6:9]
        semid = sems[9:12]
        c = lax.axis_index("c")
        s = lax.axis_index("s")
        row0 = s * WROWS
        out0 = c * NP + row0
        edge0 = c * E_PER_CORE + s * E_PER_WORKER

        def zero_accumulator():
            for z in range(NZ):
                _fill_iota(didx2.at[0], row0 + z * B)
                pltpu.sync_copy(msg2.at[0], agg_sh.at[didx2.at[0]])

        def write_out(dst_hbm_ref):
            for z in range(NZ):
                _fill_iota(didx2.at[0], row0 + z * B)
                pltpu.async_copy(agg_sh.at[didx2.at[0]], msg2.at[0],
                                 semg[0]).wait()
                pltpu.sync_copy(msg2.at[0],
                                dst_hbm_ref.at[pl.ds(out0 + z * B, B)])

        _fill_const(msg2.at[0], 0.0)
        zero_accumulator()
        plsc.subcore_barrier()

        def ifire(chunk, b):
            base = edge0 + chunk * B
            pltpu.async_copy(src_hbm.at[pl.ds(base, B)], sidx2.at[b],
                             semis[b])
            pltpu.async_copy(dst_hbm.at[pl.ds(base, B)], didx2.at[b],
                             semid[b])

        def iwait(b):
            pltpu.make_async_copy(src_hbm.at[pl.ds(0, B)], sidx2.at[b],
                                  semis[b]).wait()
            pltpu.make_async_copy(dst_hbm.at[pl.ds(0, B)], didx2.at[b],
                                  semid[b]).wait()

        def gstart(b):
            pltpu.async_copy(h_hbm.at[sidx2.at[b]], msg2.at[b], semg[b])

        def gwait(b):
            pltpu.make_async_copy(h_hbm.at[sidx2.at[b]], msg2.at[b],
                                  semg[b]).wait()

        def sstart(b):
            pltpu.async_copy(msg2.at[b], agg_sh.at[didx2.at[b]], sems_[b],
                             add=True)

        def swait(b):
            pltpu.make_async_copy(msg2.at[b], agg_sh.at[didx2.at[b]],
                                  sems_[b]).wait()

        def step(c, b, fire_i, fire_g):
            gwait(b)
            sstart(b)
            swait((b + 2) % 3)
            if fire_i:
                ifire(c + 2, (b + 2) % 3)
            if fire_g:
                iwait((b + 1) % 3)
                gstart((b + 1) % 3)

        ifire(0, 0)
        ifire(1, 1)
        iwait(0)
        gstart(0)
        gwait(0)
        sstart(0)
        ifire(2, 2)
        iwait(1)
        gstart(1)
        step(1, 1, True, True)

        def triple(q, _):
            c = 2 + 3 * q
            step(c + 0, 2, True, True)
            step(c + 1, 0, True, True)
            step(c + 2, 1, True, True)
            return 0
        lax.fori_loop(0, 40, triple, 0)
        step(122, 122 % 3, True, True)
        step(123, 123 % 3, False, True)
        step(124, 124 % 3, False, False)
        swait(124 % 3)
        plsc.subcore_barrier()
        write_out(agg_hbm)

        if not with_deg:
            return

        plsc.subcore_barrier()
        _fill_const(msg2.at[0], 0.0)
        zero_accumulator()
        plsc.subcore_barrier()
        _fill_const(msg2.at[0], 1.0)

        def dfire(chunk, b):
            base = edge0 + chunk * B
            pltpu.sync_copy(dst_hbm.at[pl.ds(base, B)], didx2.at[b])
            pltpu.async_copy(msg2.at[0], agg_sh.at[didx2.at[b]], sems_[b],
                             add=True)

        def dwait(b):
            pltpu.make_async_copy(msg2.at[0], agg_sh.at[didx2.at[b]],
                                  sems_[b]).wait()

        dfire(0, 0)
        dfire(1, 1)

        def dpair(p, _):
            for b in range(2):
                chunk = 2 * p + b
                dwait(b)
                dfire(chunk, b)
            return 0
        lax.fori_loop(1, PAIRS, dpair, 0)
        dwait(0)
        dfire(ITERS - 1, 0)
        dwait(0)
        dwait(1)
        plsc.subcore_barrier()
        write_out(deg_hbm)

    return pl.kernel(body, out_type=tuple(out_type), mesh=mesh,
                     scratch_types=scratch)


_sc_agg_deg = _sc_aggregate(with_deg=True)
_sc_agg = _sc_aggregate(with_deg=False)

R = 1000


def _tc_layer_body(relu, x_ref, agg0_ref, agg1_ref, deg0_ref, deg1_ref,
                   ws_ref, wn_ref, b_ref, out_ref):
    deg = deg0_ref[0, :, 0:1] + deg1_ref[0, :, 0:1]
    rdeg = 1.0 / jnp.maximum(deg, 1.0)
    hn = (agg0_ref[0] + agg1_ref[0]) * rdeg
    acc = jnp.dot(x_ref[...], ws_ref[...], preferred_element_type=jnp.float32)
    acc += jnp.dot(hn, wn_ref[...], preferred_element_type=jnp.float32)
    acc += b_ref[...]
    if relu:
        acc = jnp.maximum(acc, 0.0)
    out_ref[...] = acc


def _tc_layer(x, agg, deg, w_self, w_neigh, b, relu):
    grid = (N // R,)
    return pl.pallas_call(
        functools.partial(_tc_layer_body, relu),
        grid=grid,
        in_specs=[
            pl.BlockSpec((R, D), lambda i: (i, 0)),
            pl.BlockSpec((1, R, D), lambda i: (0, i, 0)),
            pl.BlockSpec((1, R, D), lambda i: (1, i, 0)),
            pl.BlockSpec((1, R, D), lambda i: (0, i, 0)),
            pl.BlockSpec((1, R, D), lambda i: (1, i, 0)),
            pl.BlockSpec((D, D), lambda i: (0, 0)),
            pl.BlockSpec((D, D), lambda i: (0, 0)),
            pl.BlockSpec((1, D), lambda i: (0, 0)),
        ],
        out_specs=pl.BlockSpec((R, D), lambda i: (i, 0)),
        out_shape=jax.ShapeDtypeStruct((N, D), jnp.float32),
    )(x, agg, agg, deg, deg, w_self, w_neigh, b)


def kernel(inputs, edge_index, W_self1, W_neigh1, b1, W_self2, W_neigh2, b2):
    src = edge_index[0]
    dst = edge_index[1]
    b1r = b1.reshape(1, D)
    b2r = b2.reshape(1, D)

    agg1, deg = _sc_agg_deg(inputs, src, dst)
    agg1 = agg1.reshape(NC, NP, D)
    deg = deg.reshape(NC, NP, D)
    h1 = _tc_layer(inputs, agg1, deg, W_self1, W_neigh1, b1r, relu=True)
    agg2, = _sc_agg(h1, src, dst)
    agg2 = agg2.reshape(NC, NP, D)
    out = _tc_layer(h1, agg2, deg, W_self2, W_neigh2, b2r, relu=False)
    return out


# speedup vs baseline: 8.8784x; 1.0067x over previous
import functools

import jax
import jax.numpy as jnp
from jax import lax
from jax.experimental import pallas as pl
from jax.experimental.pallas import tpu as pltpu
from jax.experimental.pallas import tpu_sc as plsc

N = 10000
E = 320000
D = 128

NC = 2
NS = 16
NP = 10240
E_PER_CORE = E // NC
E_PER_WORKER = E_PER_CORE // NS
B = 80
ITERS = E_PER_WORKER // B
PAIRS = (ITERS - 1) // 2
WROWS = NP // NS
NZ = WROWS // B


def _fill_iota(idx_v, base):
    lanes = lax.iota(jnp.int32, 16)
    for g in range(B // 16):
        idx_v[pl.ds(g * 16, 16)] = base + g * 16 + lanes


def _fill_const(buf_v, val):
    def init_row(i, _):
        for j in range(D // 16):
            buf_v[i, pl.ds(j * 16, 16)] = jnp.full((16,), val, jnp.float32)
        return 0
    lax.fori_loop(0, B, init_row, 0)


def _sc_aggregate(with_deg: bool):
    mesh = plsc.VectorSubcoreMesh(core_axis_name="c", subcore_axis_name="s")
    out_type = [jax.ShapeDtypeStruct((NC * NP, D), jnp.float32)]
    if with_deg:
        out_type.append(jax.ShapeDtypeStruct((NC * NP, D), jnp.float32))
    scratch = [
        pltpu.VMEM((3, B), jnp.int32),
        pltpu.VMEM((3, B), jnp.int32),
        pltpu.VMEM((3, B, D), jnp.float32),
        pltpu.VMEM_SHARED((NP, D), jnp.float32),
    ] + [pltpu.SemaphoreType.DMA] * 12

    def body(h_hbm, src_hbm, dst_hbm, *refs):
        if with_deg:
            agg_hbm, deg_hbm, sidx2, didx2, msg2, agg_sh, *sems = refs
        else:
            agg_hbm, sidx2, didx2, msg2, agg_sh, *sems = refs
        semg = sems[0:3]
        sems_ = sems[3:6]
        semis = sems[6:9]
        semid = sems[9:12]
        c = lax.axis_index("c")
        s = lax.axis_index("s")
        row0 = s * WROWS
        out0 = c * NP + row0
        edge0 = c * E_PER_CORE + s * E_PER_WORKER

        def zero_accumulator():
            for z in range(NZ):
                _fill_iota(didx2.at[0], row0 + z * B)
                pltpu.sync_copy(msg2.at[0], agg_sh.at[didx2.at[0]])

        def write_out(dst_hbm_ref):
            for z in range(NZ):
                _fill_iota(didx2.at[0], row0 + z * B)
                pltpu.async_copy(agg_sh.at[didx2.at[0]], msg2.at[0],
                                 semg[0]).wait()
                pltpu.sync_copy(msg2.at[0],
                                dst_hbm_ref.at[pl.ds(out0 + z * B, B)])

        _fill_const(msg2.at[0], 0.0)
        zero_accumulator()
        plsc.subcore_barrier()

        def ifire(chunk, b):
            base = edge0 + chunk * B
            pltpu.async_copy(src_hbm.at[pl.ds(base, B)], sidx2.at[b],
                             semis[b])
            pltpu.async_copy(dst_hbm.at[pl.ds(base, B)], didx2.at[b],
                             semid[b])

        def iwait(b):
            pltpu.make_async_copy(src_hbm.at[pl.ds(0, B)], sidx2.at[b],
                                  semis[b]).wait()
            pltpu.make_async_copy(dst_hbm.at[pl.ds(0, B)], didx2.at[b],
                                  semid[b]).wait()

        def gstart(b):
            pltpu.async_copy(h_hbm.at[sidx2.at[b]], msg2.at[b], semg[b])

        def gwait(b):
            pltpu.make_async_copy(h_hbm.at[sidx2.at[b]], msg2.at[b],
                                  semg[b]).wait()

        def sstart(b):
            pltpu.async_copy(msg2.at[b], agg_sh.at[didx2.at[b]], sems_[b],
                             add=True)

        def swait(b):
            pltpu.make_async_copy(msg2.at[b], agg_sh.at[didx2.at[b]],
                                  sems_[b]).wait()

        def step(c, b, fire_i, fire_g):
            gwait(b)
            sstart(b)
            swait((b + 2) % 3)
            if fire_i:
                ifire(c + 2, (b + 2) % 3)
            if fire_g:
                iwait((b + 1) % 3)
                gstart((b + 1) % 3)

        ifire(0, 0)
        ifire(1, 1)
        iwait(0)
        gstart(0)
        gwait(0)
        sstart(0)
        ifire(2, 2)
        iwait(1)
        gstart(1)
        step(1, 1, True, True)

        def triple(q, _):
            cc = 2 + 3 * q
            step(cc + 0, 2, True, True)
            step(cc + 1, 0, True, True)
            step(cc + 2, 1, True, True)
            return 0
        lax.fori_loop(0, 40, triple, 0)
        step(122, 122 % 3, True, True)
        step(123, 123 % 3, False, True)
        step(124, 124 % 3, False, False)
        swait(124 % 3)
        plsc.subcore_barrier()
        write_out(agg_hbm)

        if not with_deg:
            return

        plsc.subcore_barrier()
        _fill_const(msg2.at[0], 0.0)
        zero_accumulator()
        plsc.subcore_barrier()
        _fill_const(msg2.at[1], 1.0)

        def dfire(chunk, b):
            base = edge0 + chunk * B
            pltpu.async_copy(dst_hbm.at[pl.ds(base, B)], didx2.at[b],
                             semid[b])

        def diwait(b):
            pltpu.make_async_copy(dst_hbm.at[pl.ds(0, B)], didx2.at[b],
                                  semid[b]).wait()

        def dsstart(b):
            pltpu.async_copy(msg2.at[1], agg_sh.at[didx2.at[b]], sems_[b],
                             add=True)

        def dswait(b):
            pltpu.make_async_copy(msg2.at[1], agg_sh.at[didx2.at[b]],
                                  sems_[b]).wait()

        def dstep(c, b, fire_i, wait_prev=True):
            diwait(b)
            dsstart(b)
            if wait_prev:
                dswait((b + 2) % 3)
            if fire_i:
                dfire(c + 2, (b + 2) % 3)

        dfire(0, 0)
        dfire(1, 1)
        dstep(0, 0, True, wait_prev=False)
        dstep(1, 1, True)

        def dtriple(q, _):
            cc = 2 + 3 * q
            dstep(cc + 0, 2, True)
            dstep(cc + 1, 0, True)
            dstep(cc + 2, 1, True)
            return 0
        lax.fori_loop(0, 40, dtriple, 0)
        dstep(122, 122 % 3, True)
        dstep(123, 123 % 3, False)
        dstep(124, 124 % 3, False)
        dswait(124 % 3)
        plsc.subcore_barrier()
        write_out(deg_hbm)

    return pl.kernel(body, out_type=tuple(out_type), mesh=mesh,
                     scratch_types=scratch)


_sc_agg_deg = _sc_aggregate(with_deg=True)
_sc_agg = _sc_aggregate(with_deg=False)

R = 1024


def _tc_layer_body(relu, x_ref, agg0_ref, agg1_ref, deg0_ref, deg1_ref,
                   ws_ref, wn_ref, b_ref, out_ref):
    deg = deg0_ref[0, :, 0:1] + deg1_ref[0, :, 0:1]
    rdeg = 1.0 / jnp.maximum(deg, 1.0)
    hn = (agg0_ref[0] + agg1_ref[0]) * rdeg
    acc = jnp.dot(x_ref[...], ws_ref[...], preferred_element_type=jnp.float32)
    acc += jnp.dot(hn, wn_ref[...], preferred_element_type=jnp.float32)
    acc += b_ref[...]
    if relu:
        acc = jnp.maximum(acc, 0.0)
    out_ref[...] = acc


def _tc_layer(x, agg, deg, w_self, w_neigh, b, relu):
    grid = (NP // R,)
    return pl.pallas_call(
        functools.partial(_tc_layer_body, relu),
        grid=grid,
        in_specs=[
            pl.BlockSpec((R, D), lambda i: (i, 0)),
            pl.BlockSpec((1, R, D), lambda i: (0, i, 0)),
            pl.BlockSpec((1, R, D), lambda i: (1, i, 0)),
            pl.BlockSpec((1, R, D), lambda i: (0, i, 0)),
            pl.BlockSpec((1, R, D), lambda i: (1, i, 0)),
            pl.BlockSpec((D, D), lambda i: (0, 0)),
            pl.BlockSpec((D, D), lambda i: (0, 0)),
            pl.BlockSpec((1, D), lambda i: (0, 0)),
        ],
        out_specs=pl.BlockSpec((R, D), lambda i: (i, 0)),
        out_shape=jax.ShapeDtypeStruct((N, D), jnp.float32),
    )(x, agg, agg, deg, deg, w_self, w_neigh, b)


def kernel(inputs, edge_index, W_self1, W_neigh1, b1, W_self2, W_neigh2, b2):
    src = edge_index[0]
    dst = edge_index[1]
    b1r = b1.reshape(1, D)
    b2r = b2.reshape(1, D)

    agg1, deg = _sc_agg_deg(inputs, src, dst)
    agg1 = agg1.reshape(NC, NP, D)
    deg = deg.reshape(NC, NP, D)
    h1 = _tc_layer(inputs, agg1, deg, W_self1, W_neigh1, b1r, relu=True)
    agg2, = _sc_agg(h1, src, dst)
    agg2 = agg2.reshape(NC, NP, D)
    out = _tc_layer(h1, agg2, deg, W_self2, W_neigh2, b2r, relu=False)
    return out


# speedup vs baseline: 10.4920x; 1.1817x over previous
import functools

import jax
import jax.numpy as jnp
from jax import lax
from jax.experimental import pallas as pl
from jax.experimental.pallas import tpu as pltpu
from jax.experimental.pallas import tpu_sc as plsc

N = 10000
E = 320000
D = 128

NC = 2
NS = 16
NP = 10240
E_PER_CORE = E // NC
E_PER_WORKER = E_PER_CORE // NS
B = 80
ITERS = E_PER_WORKER // B
PAIRS = (ITERS - 1) // 2
WROWS = NP // NS
NZ = WROWS // B


def _fill_iota(idx_v, base):
    lanes = lax.iota(jnp.int32, 16)
    for g in range(B // 16):
        idx_v[pl.ds(g * 16, 16)] = base + g * 16 + lanes


def _fill_const(buf_v, val):
    def init_row(i, _):
        for j in range(D // 16):
            buf_v[i, pl.ds(j * 16, 16)] = jnp.full((16,), val, jnp.float32)
        return 0
    lax.fori_loop(0, B, init_row, 0)


def _sc_aggregate(with_deg: bool):
    mesh = plsc.VectorSubcoreMesh(core_axis_name="c", subcore_axis_name="s")
    out_type = [jax.ShapeDtypeStruct((NC * NP, D), jnp.float32)]
    if with_deg:
        out_type.append(jax.ShapeDtypeStruct((NC * NP, D), jnp.float32))
    scratch = [
        pltpu.VMEM((3, B), jnp.int32),
        pltpu.VMEM((3, B), jnp.int32),
        pltpu.VMEM((3, B, D), jnp.float32),
        pltpu.VMEM_SHARED((NP, D), jnp.float32),
    ] + [pltpu.SemaphoreType.DMA] * 12

    def body(h_hbm, src_hbm, dst_hbm, *refs):
        if with_deg:
            agg_hbm, deg_hbm, sidx2, didx2, msg2, agg_sh, *sems = refs
        else:
            agg_hbm, sidx2, didx2, msg2, agg_sh, *sems = refs
        semg = sems[0:3]
        sems_ = sems[3:6]
        semis = sems[6:9]
        semid = sems[9:12]
        c = lax.axis_index("c")
        s = lax.axis_index("s")
        row0 = s * WROWS
        out0 = c * NP + row0
        edge0 = c * E_PER_CORE + s * E_PER_WORKER

        def zero_accumulator():
            for z in range(NZ):
                _fill_iota(didx2.at[0], row0 + z * B)
                pltpu.sync_copy(msg2.at[0], agg_sh.at[didx2.at[0]])

        def write_out(dst_hbm_ref):
            for z in range(NZ):
                _fill_iota(didx2.at[0], row0 + z * B)
                pltpu.async_copy(agg_sh.at[didx2.at[0]], msg2.at[0],
                                 semg[0]).wait()
                pltpu.sync_copy(msg2.at[0],
                                dst_hbm_ref.at[pl.ds(out0 + z * B, B)])

        _fill_const(msg2.at[0], 0.0)
        zero_accumulator()
        plsc.subcore_barrier()

        def ifire(chunk, b):
            base = edge0 + chunk * B
            pltpu.async_copy(src_hbm.at[pl.ds(base, B)], sidx2.at[b],
                             semis[b])
            pltpu.async_copy(dst_hbm.at[pl.ds(base, B)], didx2.at[b],
                             semid[b])

        def iwait(b):
            pltpu.make_async_copy(src_hbm.at[pl.ds(0, B)], sidx2.at[b],
                                  semis[b]).wait()
            pltpu.make_async_copy(dst_hbm.at[pl.ds(0, B)], didx2.at[b],
                                  semid[b]).wait()

        def gstart(b):
            pltpu.async_copy(h_hbm.at[sidx2.at[b]], msg2.at[b], semg[b])

        def gwait(b):
            pltpu.make_async_copy(h_hbm.at[sidx2.at[b]], msg2.at[b],
                                  semg[b]).wait()

        def sstart(b):
            pltpu.async_copy(msg2.at[b], agg_sh.at[didx2.at[b]], sems_[b],
                             add=True)

        def swait(b):
            pltpu.make_async_copy(msg2.at[b], agg_sh.at[didx2.at[b]],
                                  sems_[b]).wait()

        def step(c, b, fire_i, fire_g, wait_prev=True):
            if fire_g:
                iwait((b + 1) % 3)
                gstart((b + 1) % 3)
            gwait(b)
            sstart(b)
            if wait_prev:
                swait((b + 2) % 3)
            if fire_i:
                ifire(c + 2, (b + 2) % 3)

        ifire(0, 0)
        ifire(1, 1)
        iwait(0)
        gstart(0)
        ifire(2, 2)
        step(0, 0, False, True, wait_prev=False)
        step(1, 1, True, True)

        def triple(q, _):
            cc = 2 + 3 * q
            step(cc + 0, 2, True, True)
            step(cc + 1, 0, True, True)
            step(cc + 2, 1, True, True)
            return 0
        lax.fori_loop(0, 40, triple, 0)
        step(122, 122 % 3, True, True)
        step(123, 123 % 3, False, True)
        step(124, 124 % 3, False, False)
        swait(124 % 3)
        plsc.subcore_barrier()
        write_out(agg_hbm)

        if not with_deg:
            return

        plsc.subcore_barrier()
        _fill_const(msg2.at[0], 0.0)
        zero_accumulator()
        plsc.subcore_barrier()
        _fill_const(msg2.at[1], 1.0)

        def dfire(chunk, b):
            base = edge0 + chunk * B
            pltpu.async_copy(dst_hbm.at[pl.ds(base, B)], didx2.at[b],
                             semid[b])

        def diwait(b):
            pltpu.make_async_copy(dst_hbm.at[pl.ds(0, B)], didx2.at[b],
                                  semid[b]).wait()

        def dsstart(b):
            pltpu.async_copy(msg2.at[1], agg_sh.at[didx2.at[b]], sems_[b],
                             add=True)

        def dswait(b):
            pltpu.make_async_copy(msg2.at[1], agg_sh.at[didx2.at[b]],
                                  sems_[b]).wait()

        def dstep(c, b, fire_i, wait_prev=True):
            diwait(b)
            dsstart(b)
            if wait_prev:
                dswait((b + 2) % 3)
            if fire_i:
                dfire(c + 2, (b + 2) % 3)

        dfire(0, 0)
        dfire(1, 1)
        dstep(0, 0, True, wait_prev=False)
        dstep(1, 1, True)

        def dtriple(q, _):
            cc = 2 + 3 * q
            dstep(cc + 0, 2, True)
            dstep(cc + 1, 0, True)
            dstep(cc + 2, 1, True)
            return 0
        lax.fori_loop(0, 40, dtriple, 0)
        dstep(122, 122 % 3, True)
        dstep(123, 123 % 3, False)
        dstep(124, 124 % 3, False)
        dswait(124 % 3)
        plsc.subcore_barrier()
        write_out(deg_hbm)

    return pl.kernel(body, out_type=tuple(out_type), mesh=mesh,
                     scratch_types=scratch)


_sc_agg_deg = _sc_aggregate(with_deg=True)
_sc_agg = _sc_aggregate(with_deg=False)

R = 1024


def _tc_layer_body(relu, x_ref, agg0_ref, agg1_ref, deg0_ref, deg1_ref,
                   ws_ref, wn_ref, b_ref, out_ref):
    deg = deg0_ref[0, :, 0:1] + deg1_ref[0, :, 0:1]
    rdeg = 1.0 / jnp.maximum(deg, 1.0)
    hn = (agg0_ref[0] + agg1_ref[0]) * rdeg
    acc = jnp.dot(x_ref[...], ws_ref[...], preferred_element_type=jnp.float32)
    acc += jnp.dot(hn, wn_ref[...], preferred_element_type=jnp.float32)
    acc += b_ref[...]
    if relu:
        acc = jnp.maximum(acc, 0.0)
    out_ref[...] = acc


def _tc_layer(x, agg, deg, w_self, w_neigh, b, relu):
    grid = (NP // R,)
    return pl.pallas_call(
        functools.partial(_tc_layer_body, relu),
        grid=grid,
        in_specs=[
            pl.BlockSpec((R, D), lambda i: (i, 0)),
            pl.BlockSpec((1, R, D), lambda i: (0, i, 0)),
            pl.BlockSpec((1, R, D), lambda i: (1, i, 0)),
            pl.BlockSpec((1, R, D), lambda i: (0, i, 0)),
            pl.BlockSpec((1, R, D), lambda i: (1, i, 0)),
            pl.BlockSpec((D, D), lambda i: (0, 0)),
            pl.BlockSpec((D, D), lambda i: (0, 0)),
            pl.BlockSpec((1, D), lambda i: (0, 0)),
        ],
        out_specs=pl.BlockSpec((R, D), lambda i: (i, 0)),
        out_shape=jax.ShapeDtypeStruct((N, D), jnp.float32),
    )(x, agg, agg, deg, deg, w_self, w_neigh, b)


def kernel(inputs, edge_index, W_self1, W_neigh1, b1, W_self2, W_neigh2, b2):
    src = edge_index[0]
    dst = edge_index[1]
    b1r = b1.reshape(1, D)
    b2r = b2.reshape(1, D)

    agg1, deg = _sc_agg_deg(inputs, src, dst)
    agg1 = agg1.reshape(NC, NP, D)
    deg = deg.reshape(NC, NP, D)
    h1 = _tc_layer(inputs, agg1, deg, W_self1, W_neigh1, b1r, relu=True)
    agg2, = _sc_agg(h1, src, dst)
    agg2 = agg2.reshape(NC, NP, D)
    out = _tc_layer(h1, agg2, deg, W_self2, W_neigh2, b2r, relu=False)
    return out


# speedup vs baseline: 10.4969x; 1.0005x over previous
import functools

import jax
import jax.numpy as jnp
from jax import lax
from jax.experimental import pallas as pl
from jax.experimental.pallas import tpu as pltpu
from jax.experimental.pallas import tpu_sc as plsc

N = 10000
E = 320000
D = 128

NC = 2
NS = 16
NP = 10240
E_PER_CORE = E // NC
E_PER_WORKER = E_PER_CORE // NS
B = 80
ITERS = E_PER_WORKER // B
WROWS = NP // NS
NZ = WROWS // B


def _fill_iota(idx_v, base):
    lanes = lax.iota(jnp.int32, 16)
    for g in range(B // 16):
        idx_v[pl.ds(g * 16, 16)] = base + g * 16 + lanes


def _fill_const(buf_v, val):
    def init_row(i, _):
        for j in range(D // 16):
            buf_v[i, pl.ds(j * 16, 16)] = jnp.full((16,), val, jnp.float32)
        return 0
    lax.fori_loop(0, B, init_row, 0)


def _sc_aggregate(with_deg: bool):
    mesh = plsc.VectorSubcoreMesh(core_axis_name="c", subcore_axis_name="s")
    out_type = [jax.ShapeDtypeStruct((NC * NP, D), jnp.float32)]
    if with_deg:
        out_type.append(jax.ShapeDtypeStruct((NC * NP, D), jnp.float32))
    scratch = [
        pltpu.VMEM((3, B), jnp.int32),
        pltpu.VMEM((3, B), jnp.int32),
        pltpu.VMEM((3, B, D), jnp.float32),
        pltpu.VMEM_SHARED((NP, D), jnp.float32),
    ] + [pltpu.SemaphoreType.DMA] * 12

    def body(h_hbm, src_hbm, dst_hbm, *refs):
        if with_deg:
            agg_hbm, deg_hbm, sidx2, didx2, msg2, agg_sh, *sems = refs
        else:
            agg_hbm, sidx2, didx2, msg2, agg_sh, *sems = refs
        semg = sems[0:3]
        sems_ = sems[3:6]
        semis = sems[6:9]
        semid = sems[9:12]
        c = lax.axis_index("c")
        s = lax.axis_index("s")
        row0 = s * WROWS
        out0 = c * NP + row0
        edge0 = c * E_PER_CORE + s * E_PER_WORKER

        def zero_accumulator():
            for z in range(NZ):
                _fill_iota(didx2.at[0], row0 + z * B)
                pltpu.sync_copy(msg2.at[0], agg_sh.at[didx2.at[0]])

        def write_out(dst_hbm_ref):
            for z in range(NZ):
                _fill_iota(didx2.at[0], row0 + z * B)
                pltpu.async_copy(agg_sh.at[didx2.at[0]], msg2.at[0],
                                 semg[0]).wait()
                pltpu.sync_copy(msg2.at[0],
                                dst_hbm_ref.at[pl.ds(out0 + z * B, B)])

        _fill_const(msg2.at[0], 0.0)
        zero_accumulator()
        plsc.subcore_barrier()

        def ifire(chunk, b):
            base = edge0 + chunk * B
            pltpu.async_copy(src_hbm.at[pl.ds(base, B)], sidx2.at[b],
                             semis[b])
            pltpu.async_copy(dst_hbm.at[pl.ds(base, B)], didx2.at[b],
                             semid[b])

        def iwait(b):
            pltpu.make_async_copy(src_hbm.at[pl.ds(0, B)], sidx2.at[b],
                                  semis[b]).wait()
            pltpu.make_async_copy(dst_hbm.at[pl.ds(0, B)], didx2.at[b],
                                  semid[b]).wait()

        def gstart(b):
            pltpu.async_copy(h_hbm.at[sidx2.at[b]], msg2.at[b], semg[b])

        def gwait(b):
            pltpu.make_async_copy(h_hbm.at[sidx2.at[b]], msg2.at[b],
                                  semg[b]).wait()

        def sstart(b):
            pltpu.async_copy(msg2.at[b], agg_sh.at[didx2.at[b]], sems_[b],
                             add=True)

        def swait(b):
            pltpu.make_async_copy(msg2.at[b], agg_sh.at[didx2.at[b]],
                                  sems_[b]).wait()

        def step(c, b, fire_i, fire_g, wait_prev=True):
            if fire_g:
                iwait((b + 1) % 3)
                gstart((b + 1) % 3)
            gwait(b)
            sstart(b)
            if wait_prev:
                swait((b + 2) % 3)
            if fire_i:
                ifire(c + 2, (b + 2) % 3)

        ifire(0, 0)
        ifire(1, 1)
        iwait(0)
        gstart(0)
        ifire(2, 2)
        step(0, 0, False, True, wait_prev=False)
        step(1, 1, True, True)

        def triple(q, _):
            cc = 2 + 3 * q
            step(cc + 0, 2, True, True)
            step(cc + 1, 0, True, True)
            step(cc + 2, 1, True, True)
            return 0
        lax.fori_loop(0, 40, triple, 0)
        step(122, 122 % 3, True, True)
        step(123, 123 % 3, False, True)
        step(124, 124 % 3, False, False)
        swait(124 % 3)
        plsc.subcore_barrier()
        write_out(agg_hbm)

        if not with_deg:
            return

        plsc.subcore_barrier()
        _fill_const(msg2.at[0], 0.0)
        zero_accumulator()
        plsc.subcore_barrier()
        _fill_const(msg2.at[1], 1.0)

        def dfire(chunk, b):
            base = edge0 + chunk * B
            pltpu.async_copy(dst_hbm.at[pl.ds(base, B)], didx2.at[b],
                             semid[b])

        def diwait(b):
            pltpu.make_async_copy(dst_hbm.at[pl.ds(0, B)], didx2.at[b],
                                  semid[b]).wait()

        def dsstart(b):
            pltpu.async_copy(msg2.at[1], agg_sh.at[didx2.at[b]], sems_[b],
                             add=True)

        def dswait(b):
            pltpu.make_async_copy(msg2.at[1], agg_sh.at[didx2.at[b]],
                                  sems_[b]).wait()

        def dstep(c, b, fire_i, wait_prev=True):
            diwait(b)
            dsstart(b)
            if wait_prev:
                dswait((b + 2) % 3)
            if fire_i:
                dfire(c + 2, (b + 2) % 3)

        dfire(0, 0)
        dfire(1, 1)
        dstep(0, 0, True, wait_prev=False)
        dstep(1, 1, True)

        def dtriple(q, _):
            cc = 2 + 3 * q
            dstep(cc + 0, 2, True)
            dstep(cc + 1, 0, True)
            dstep(cc + 2, 1, True)
            return 0
        lax.fori_loop(0, 40, dtriple, 0)
        dstep(122, 122 % 3, True)
        dstep(123, 123 % 3, False)
        dstep(124, 124 % 3, False)
        dswait(124 % 3)
        plsc.subcore_barrier()
        write_out(deg_hbm)

    return pl.kernel(body, out_type=tuple(out_type), mesh=mesh,
                     scratch_types=scratch)


_sc_agg_deg = _sc_aggregate(with_deg=True)
_sc_agg = _sc_aggregate(with_deg=False)

R = 1024


def _tc_layer_body(relu, x_ref, agg0_ref, agg1_ref, deg0_ref, deg1_ref,
                   ws_ref, wn_ref, b_ref, out_ref):
    deg = deg0_ref[0, :, 0:1] + deg1_ref[0, :, 0:1]
    rdeg = 1.0 / jnp.maximum(deg, 1.0)
    hn = (agg0_ref[0] + agg1_ref[0]) * rdeg
    acc = jnp.dot(x_ref[...], ws_ref[...], preferred_element_type=jnp.float32)
    acc += jnp.dot(hn, wn_ref[...], preferred_element_type=jnp.float32)
    acc += b_ref[...]
    if relu:
        acc = jnp.maximum(acc, 0.0)
    out_ref[...] = acc


def _tc_layer(x, agg, deg, w_self, w_neigh, b, relu):
    grid = (NP // R,)
    return pl.pallas_call(
        functools.partial(_tc_layer_body, relu),
        grid=grid,
        in_specs=[
            pl.BlockSpec((R, D), lambda i: (i, 0)),
            pl.BlockSpec((1, R, D), lambda i: (0, i, 0)),
            pl.BlockSpec((1, R, D), lambda i: (1, i, 0)),
            pl.BlockSpec((1, R, D), lambda i: (0, i, 0)),
            pl.BlockSpec((1, R, D), lambda i: (1, i, 0)),
            pl.BlockSpec((D, D), lambda i: (0, 0)),
            pl.BlockSpec((D, D), lambda i: (0, 0)),
            pl.BlockSpec((1, D), lambda i: (0, 0)),
        ],
        out_specs=pl.BlockSpec((R, D), lambda i: (i, 0)),
        out_shape=jax.ShapeDtypeStruct((N, D), jnp.float32),
    )(x, agg, agg, deg, deg, w_self, w_neigh, b)


def kernel(inputs, edge_index, W_self1, W_neigh1, b1, W_self2, W_neigh2, b2):
    src = edge_index[0]
    dst = edge_index[1]
    b1r = b1.reshape(1, D)
    b2r = b2.reshape(1, D)

    agg1, deg = _sc_agg_deg(inputs, src, dst)
    agg1 = agg1.reshape(NC, NP, D)
    deg = deg.reshape(NC, NP, D)
    h1 = _tc_layer(inputs, agg1, deg, W_self1, W_neigh1, b1r, relu=True)
    agg2, = _sc_agg(h1, src, dst)
    agg2 = agg2.reshape(NC, NP, D)
    out = _tc_layer(h1, agg2, deg, W_self2, W_neigh2, b2r, relu=False)
    return out
